# Initial kernel scaffold; baseline (speedup 1.0000x reference)
import jax
import jax.numpy as jnp
from jax.experimental import pallas as pl


def kernel(x, edge_index, edge_attr, W, We, att, bias):
    raise NotImplementedError("write your pallas kernel here")


# speedup vs baseline: 10.1941x; 10.1941x over previous
import functools

import jax
import jax.numpy as jnp
from jax import lax
from jax.experimental import pallas as pl
from jax.experimental.pallas import tpu as pltpu
from jax.experimental.pallas import tpu_sc as plsc

N_NODES = 10000
N_EDGES = 320000
CH = 128
NEG_SLOPE = 0.2

NUM_CORES = 2
NUM_SUBCORES = 16
NW = NUM_CORES * NUM_SUBCORES
E_PER = N_EDGES // NW
CHUNK = 80
NCHUNK = E_PER // CHUNK
N_PAD = 10112
N_PER = N_PAD // NUM_SUBCORES

_f32 = jnp.float32
_i32 = jnp.int32


def _k1_body(x_ref, w_ref, a2_ref, xw_ref, s_ref, smax_ref):
    xw = jnp.dot(x_ref[...], w_ref[...], preferred_element_type=_f32)
    xw_ref[...] = xw
    s = jnp.dot(xw, a2_ref[...], preferred_element_type=_f32)
    s_ref[...] = s
    smax_ref[...] = jnp.max(s, axis=0, keepdims=True)


_k1 = pl.pallas_call(
    _k1_body,
    out_shape=(
        jax.ShapeDtypeStruct((N_NODES, CH), _f32),
        jax.ShapeDtypeStruct((N_NODES, 8), _f32),
        jax.ShapeDtypeStruct((1, 8), _f32),
    ),
)

_K2_B = E_PER
_K2_G = N_EDGES // _K2_B


def _k2_body(eattr_ref, we_ref, ae_ref, ea_ref, se_ref, semax_ref):
    ea = jnp.dot(eattr_ref[...], we_ref[...], preferred_element_type=_f32)
    ea_ref[...] = ea
    se = jnp.sum(ea * ae_ref[...], axis=-1)
    se_ref[...] = se.reshape(1, 1, _K2_B)
    semax_ref[...] = jnp.full((1, 1, 8), jnp.max(se), dtype=_f32)


_k2 = pl.pallas_call(
    _k2_body,
    grid=(_K2_G,),
    in_specs=[
        pl.BlockSpec((_K2_B, 16), lambda i: (i, 0)),
        pl.BlockSpec((16, 4), lambda i: (0, 0)),
        pl.BlockSpec((1, 4), lambda i: (0, 0)),
    ],
    out_specs=[
        pl.BlockSpec((_K2_B, 4), lambda i: (i, 0)),
        pl.BlockSpec((1, 1, _K2_B), lambda i: (i, 0, 0)),
        pl.BlockSpec((1, 1, 8), lambda i: (i, 0, 0)),
    ],
    out_shape=(
        jax.ShapeDtypeStruct((N_EDGES, 4), _f32),
        jax.ShapeDtypeStruct((_K2_G, 1, _K2_B), _f32),
        jax.ShapeDtypeStruct((_K2_G, 1, 8), _f32),
    ),
)


def _k7_body(p_ref, dpt_ref, b_ref, o_ref):
    den = dpt_ref[:, 0:1] + dpt_ref[:, 1:2] + 1e-16
    o_ref[...] = (p_ref[0, :N_NODES] + p_ref[1, :N_NODES]) / den + b_ref[...]


_k7 = pl.pallas_call(
    _k7_body,
    out_shape=jax.ShapeDtypeStruct((N_NODES, CH), _f32),
)


_SC_MESH = plsc.VectorSubcoreMesh(core_axis_name="c", subcore_axis_name="s")


def _attn_body(si_hbm, sj_hbm, se_hbm, src_hbm, dst_hbm, m_hbm, zn_hbm,
               w_hbm, dpart_hbm,
               si_v, sj_v, se_v, src_v, dst_v, w_v, m_v, den_sh):
    c = lax.axis_index("c")
    s = lax.axis_index("s")
    wid = c * NUM_SUBCORES + s

    @pl.when(s == 0)
    def _():
        pltpu.sync_copy(zn_hbm, den_sh)

    pltpu.sync_copy(si_hbm, si_v)
    pltpu.sync_copy(sj_hbm, sj_v)
    pltpu.sync_copy(se_hbm.at[wid, 0], se_v)
    pltpu.sync_copy(src_hbm.at[wid], src_v)
    pltpu.sync_copy(dst_hbm.at[wid], dst_v)
    pltpu.sync_copy(m_hbm, m_v)
    gmax = m_v[...]
    plsc.subcore_barrier()

    def jbody(j, carry):
        for g in range(CHUNK // 16):
            sl = pl.ds(g * 16, 16)
            di = dst_v[j, sl]
            sri = src_v[j, sl]
            l = (plsc.load_gather(si_v, [di])
                 + plsc.load_gather(sj_v, [sri])
                 + se_v[pl.ds(j * CHUNK + g * 16, 16)])
            l = jnp.where(l >= 0.0, l, l * NEG_SLOPE)
            w_v[j, sl] = jnp.exp(l - gmax)
        pltpu.sync_copy(w_v.at[j], den_sh.at[dst_v.at[j]], add=True)
        return carry

    lax.fori_loop(0, NCHUNK, jbody, 0)
    pltpu.sync_copy(w_v, w_hbm.at[wid])
    plsc.subcore_barrier()

    @pl.when(s == 0)
    def _():
        pltpu.sync_copy(den_sh, dpart_hbm.at[c])


_attn = functools.partial(
    pl.kernel,
    out_type=(
        jax.ShapeDtypeStruct((NW, NCHUNK, CHUNK), _f32),
        jax.ShapeDtypeStruct((NUM_CORES, N_NODES), _f32),
    ),
    mesh=_SC_MESH,
    compiler_params=pltpu.CompilerParams(needs_layout_passes=False),
    scratch_types=[
        pltpu.VMEM((N_NODES,), _f32),
        pltpu.VMEM((N_NODES,), _f32),
        pltpu.VMEM((E_PER,), _f32),
        pltpu.VMEM((NCHUNK, CHUNK), _i32),
        pltpu.VMEM((NCHUNK, CHUNK), _i32),
        pltpu.VMEM((NCHUNK, CHUNK), _f32),
        pltpu.VMEM((16,), _f32),
        pltpu.VMEM_SHARED((N_NODES,), _f32),
    ],
)(_attn_body)


def _aggr_body(xw_hbm, src_hbm, dst_hbm, w_hbm, zr_hbm,
               p_hbm,
               src_v, dst_v, w_c, rows_v, aggr_sh, sem):
    c = lax.axis_index("c")
    s = lax.axis_index("s")
    wid = c * NUM_SUBCORES + s

    pltpu.sync_copy(zr_hbm, aggr_sh.at[pl.ds(s * N_PER, N_PER)])
    pltpu.sync_copy(src_hbm.at[wid], src_v)
    pltpu.sync_copy(dst_hbm.at[wid], dst_v)
    plsc.subcore_barrier()

    def jbody(j, carry):
        pltpu.sync_copy(w_hbm.at[wid, j], w_c)
        pltpu.async_copy(xw_hbm.at[src_v.at[j]], rows_v, sem).wait()

        def ibody(i, icarry):
            a = plsc.load_gather(w_c, [jnp.full((16,), i, _i32)])
            for f in range(CH // 16):
                fl = pl.ds(f * 16, 16)
                rows_v[i, fl] = rows_v[i, fl] * a
            return icarry

        lax.fori_loop(0, CHUNK, ibody, 0)
        pltpu.sync_copy(rows_v, aggr_sh.at[dst_v.at[j]], add=True)
        return carry

    lax.fori_loop(0, NCHUNK, jbody, 0)
    plsc.subcore_barrier()
    pltpu.sync_copy(aggr_sh.at[pl.ds(s * N_PER, N_PER)],
                    p_hbm.at[c, pl.ds(s * N_PER, N_PER)])


_aggr = functools.partial(
    pl.kernel,
    out_type=jax.ShapeDtypeStruct((NUM_CORES, N_PAD, CH), _f32),
    mesh=_SC_MESH,
    compiler_params=pltpu.CompilerParams(needs_layout_passes=False),
    scratch_types=[
        pltpu.VMEM((NCHUNK, CHUNK), _i32),
        pltpu.VMEM((NCHUNK, CHUNK), _i32),
        pltpu.VMEM((CHUNK,), _f32),
        pltpu.VMEM((CHUNK, CH), _f32),
        pltpu.VMEM_SHARED((N_PAD, CH), _f32),
        pltpu.SemaphoreType.DMA,
    ],
)(_aggr_body)


@jax.jit
def kernel(x, edge_index, edge_attr, W, We, att, bias):
    src = edge_index[0].astype(_i32).reshape(NW, NCHUNK, CHUNK)
    dst = edge_index[1].astype(_i32).reshape(NW, NCHUNK, CHUNK)
    attf = att.reshape(2 * CH + 4)
    a2 = jnp.pad(jnp.stack([attf[:CH], attf[CH:2 * CH]], axis=1),
                 ((0, 0), (0, 6)))
    ae = attf[2 * CH:].reshape(1, 4)

    xw, s, smax = _k1(x, W, a2)
    ea, se, semax = _k2(edge_attr, We, ae)
    s_i = s[:, 0]
    s_j = s[:, 1]

    t = smax[0, 0] + smax[0, 1] + jnp.max(semax[:, 0, 0])
    m = jnp.where(t >= 0.0, t, NEG_SLOPE * t)
    m_arr = jnp.full((16,), m, dtype=_f32)
    zn = jnp.zeros((N_NODES,), dtype=_f32)
    zr = jnp.zeros((N_PER, CH), dtype=_f32)

    w2d, dpart = _attn(s_i, s_j, se, src, dst, m_arr, zn)
    parts = _aggr(xw, src, dst, w2d, zr)
    out = _k7(parts, dpart.T, bias.reshape(1, CH))
    return out, edge_index, ea


# speedup vs baseline: 15.2950x; 1.5004x over previous
import functools

import jax
import jax.numpy as jnp
from jax import lax
from jax.experimental import pallas as pl
from jax.experimental.pallas import tpu as pltpu
from jax.experimental.pallas import tpu_sc as plsc

N_NODES = 10000
N_EDGES = 320000
CH = 128
NEG_SLOPE = 0.2

NUM_CORES = 2
NUM_SUBCORES = 16
NW = NUM_CORES * NUM_SUBCORES
E_PER = N_EDGES // NW
CHUNK = 80
NCHUNK = E_PER // CHUNK
N_PAD = 10112
N_PER = N_PAD // NUM_SUBCORES

_f32 = jnp.float32
_i32 = jnp.int32


def _k1_body(x_ref, w_ref, a2_ref, xw_ref, s_ref, smax_ref):
    xw = jnp.dot(x_ref[...], w_ref[...], preferred_element_type=_f32)
    xw_ref[...] = xw
    s = jnp.dot(xw, a2_ref[...], preferred_element_type=_f32)
    s_ref[...] = s
    smax_ref[...] = jnp.max(s, axis=0, keepdims=True)


_k1 = pl.pallas_call(
    _k1_body,
    out_shape=(
        jax.ShapeDtypeStruct((N_NODES, CH), _f32),
        jax.ShapeDtypeStruct((N_NODES, 8), _f32),
        jax.ShapeDtypeStruct((1, 8), _f32),
    ),
)

def _k2_body(eat_ref, wet_ref, ae_ref, eat_out_ref, se_ref, semax_ref):
    eat = jnp.dot(wet_ref[...], eat_ref[...], preferred_element_type=_f32)
    eat_out_ref[...] = eat[:4]
    se = jnp.sum(eat * ae_ref[...], axis=0)
    se_ref[...] = se
    semax_ref[...] = jnp.full((1, 8), jnp.max(se), dtype=_f32)


_k2 = pl.pallas_call(
    _k2_body,
    out_shape=(
        jax.ShapeDtypeStruct((4, N_EDGES), _f32),
        jax.ShapeDtypeStruct((N_EDGES,), _f32),
        jax.ShapeDtypeStruct((1, 8), _f32),
    ),
)


def _k7_body(p_ref, dpt_ref, b_ref, o_ref):
    den = dpt_ref[:, 0:1] + dpt_ref[:, 1:2] + 1e-16
    o_ref[...] = (p_ref[0, :N_NODES] + p_ref[1, :N_NODES]) / den + b_ref[...]


_k7 = pl.pallas_call(
    _k7_body,
    out_shape=jax.ShapeDtypeStruct((N_NODES, CH), _f32),
)


_SC_MESH = plsc.VectorSubcoreMesh(core_axis_name="c", subcore_axis_name="s")


def _attn_body(si_hbm, sj_hbm, se_hbm, src_hbm, dst_hbm, m_hbm, zn_hbm,
               w_hbm, dpart_hbm,
               si_v, sj_v, se_v, src_v, dst_v, w_v, m_v, den_sh):
    c = lax.axis_index("c")
    s = lax.axis_index("s")
    wid = c * NUM_SUBCORES + s

    @pl.when(s == 0)
    def _():
        pltpu.sync_copy(zn_hbm, den_sh)

    pltpu.sync_copy(si_hbm, si_v)
    pltpu.sync_copy(sj_hbm, sj_v)
    pltpu.sync_copy(se_hbm.at[pl.ds(wid * E_PER, E_PER)], se_v)
    pltpu.sync_copy(src_hbm.at[wid], src_v)
    pltpu.sync_copy(dst_hbm.at[wid], dst_v)
    pltpu.sync_copy(m_hbm, m_v)
    gmax = m_v[...]
    plsc.subcore_barrier()

    def jbody(j, carry):
        for g in range(CHUNK // 16):
            sl = pl.ds(g * 16, 16)
            di = dst_v[j, sl]
            sri = src_v[j, sl]
            l = (plsc.load_gather(si_v, [di])
                 + plsc.load_gather(sj_v, [sri])
                 + se_v[pl.ds(j * CHUNK + g * 16, 16)])
            l = jnp.where(l >= 0.0, l, l * NEG_SLOPE)
            w_v[j, sl] = jnp.exp(l - gmax)
        pltpu.sync_copy(w_v.at[j], den_sh.at[dst_v.at[j]], add=True)
        return carry

    lax.fori_loop(0, NCHUNK, jbody, 0)
    pltpu.sync_copy(w_v, w_hbm.at[wid])
    plsc.subcore_barrier()

    @pl.when(s == 0)
    def _():
        pltpu.sync_copy(den_sh, dpart_hbm.at[c])


_attn = functools.partial(
    pl.kernel,
    out_type=(
        jax.ShapeDtypeStruct((NW, NCHUNK, CHUNK), _f32),
        jax.ShapeDtypeStruct((NUM_CORES, N_NODES), _f32),
    ),
    mesh=_SC_MESH,
    compiler_params=pltpu.CompilerParams(needs_layout_passes=False),
    scratch_types=[
        pltpu.VMEM((N_NODES,), _f32),
        pltpu.VMEM((N_NODES,), _f32),
        pltpu.VMEM((E_PER,), _f32),
        pltpu.VMEM((NCHUNK, CHUNK), _i32),
        pltpu.VMEM((NCHUNK, CHUNK), _i32),
        pltpu.VMEM((NCHUNK, CHUNK), _f32),
        pltpu.VMEM((16,), _f32),
        pltpu.VMEM_SHARED((N_NODES,), _f32),
    ],
)(_attn_body)


def _aggr_body(xw_hbm, src_hbm, dst_hbm, w_hbm, zr_hbm,
               p_hbm,
               src_v, dst_v, w_c, rows_v, aggr_sh, sem):
    c = lax.axis_index("c")
    s = lax.axis_index("s")
    wid = c * NUM_SUBCORES + s

    pltpu.sync_copy(zr_hbm, aggr_sh.at[pl.ds(s * N_PER, N_PER)])
    pltpu.sync_copy(src_hbm.at[wid], src_v)
    pltpu.sync_copy(dst_hbm.at[wid], dst_v)
    plsc.subcore_barrier()

    def jbody(j, carry):
        pltpu.sync_copy(w_hbm.at[wid, j], w_c)
        pltpu.async_copy(xw_hbm.at[src_v.at[j]], rows_v, sem).wait()

        def ibody(i, icarry):
            a = plsc.load_gather(w_c, [jnp.full((16,), i, _i32)])
            for f in range(CH // 16):
                fl = pl.ds(f * 16, 16)
                rows_v[i, fl] = rows_v[i, fl] * a
            return icarry

        lax.fori_loop(0, CHUNK, ibody, 0)
        pltpu.sync_copy(rows_v, aggr_sh.at[dst_v.at[j]], add=True)
        return carry

    lax.fori_loop(0, NCHUNK, jbody, 0)
    plsc.subcore_barrier()
    pltpu.sync_copy(aggr_sh.at[pl.ds(s * N_PER, N_PER)],
                    p_hbm.at[c, pl.ds(s * N_PER, N_PER)])


_aggr = functools.partial(
    pl.kernel,
    out_type=jax.ShapeDtypeStruct((NUM_CORES, N_PAD, CH), _f32),
    mesh=_SC_MESH,
    compiler_params=pltpu.CompilerParams(needs_layout_passes=False),
    scratch_types=[
        pltpu.VMEM((NCHUNK, CHUNK), _i32),
        pltpu.VMEM((NCHUNK, CHUNK), _i32),
        pltpu.VMEM((CHUNK,), _f32),
        pltpu.VMEM((CHUNK, CH), _f32),
        pltpu.VMEM_SHARED((N_PAD, CH), _f32),
        pltpu.SemaphoreType.DMA,
    ],
)(_aggr_body)


@jax.jit
def kernel(x, edge_index, edge_attr, W, We, att, bias):
    src = edge_index[0].astype(_i32).reshape(NW, NCHUNK, CHUNK)
    dst = edge_index[1].astype(_i32).reshape(NW, NCHUNK, CHUNK)
    attf = att.reshape(2 * CH + 4)
    a2 = jnp.pad(jnp.stack([attf[:CH], attf[CH:2 * CH]], axis=1),
                 ((0, 0), (0, 6)))
    wet = jnp.pad(We.T, ((0, 4), (0, 0)))
    ae = jnp.pad(attf[2 * CH:].reshape(4, 1), ((0, 4), (0, 0)))

    xw, s, smax = _k1(x, W, a2)
    eat, se, semax = _k2(edge_attr.T, wet, ae)
    ea = eat.T
    s_i = s[:, 0]
    s_j = s[:, 1]

    t = smax[0, 0] + smax[0, 1] + semax[0, 0]
    m = jnp.where(t >= 0.0, t, NEG_SLOPE * t)
    m_arr = jnp.full((16,), m, dtype=_f32)
    zn = jnp.zeros((N_NODES,), dtype=_f32)
    zr = jnp.zeros((N_PER, CH), dtype=_f32)

    w2d, dpart = _attn(s_i, s_j, se, src, dst, m_arr, zn)
    parts = _aggr(xw, src, dst, w2d, zr)
    out = _k7(parts, dpart.T, bias.reshape(1, CH))
    return out, edge_index, ea
